# baseline (speedup 1.0000x reference)
import functools

import jax
import jax.numpy as jnp
from jax import lax
from jax.experimental import pallas as pl
from jax.experimental.pallas import tpu as pltpu
from jax.experimental.pallas import tpu_sc as plsc

_BUF_ROWS = (32, 24)
_CHUNK_ROWS = (16, 24, 32, 24, 32)


def _chunk_schedule(rows_per_w: int):
    if rows_per_w == sum(_CHUNK_ROWS):
        sizes = _CHUNK_ROWS
    else:
        sizes = []
        off = 0
        while off < rows_per_w:
            sizes.append(min(_BUF_ROWS[len(sizes) % 2], rows_per_w - off))
            off += sizes[-1]
    sched = []
    off = 0
    for n in sizes:
        sched.append((off, n))
        off += n
    return sched


@functools.lru_cache(maxsize=None)
def _make_copy(seq: int, d: int, dtype_name: str):
    dtype = jnp.dtype(dtype_name)
    info = plsc.get_sparse_core_info()
    nc, ns = info.num_cores, info.num_subcores
    nw = nc * ns
    assert seq % nw == 0
    rows_per_w = seq // nw
    sched = _chunk_schedule(rows_per_w)
    nchunks = len(sched)

    mesh = plsc.VectorSubcoreMesh(core_axis_name="c", subcore_axis_name="s")

    @functools.partial(
        pl.kernel,
        mesh=mesh,
        out_type=jax.ShapeDtypeStruct((seq, d), dtype),
        scratch_types=[
            pltpu.VMEM((_BUF_ROWS[0], d), dtype),
            pltpu.VMEM((_BUF_ROWS[1], d), dtype),
            pltpu.SemaphoreType.DMA((2,)),
            pltpu.SemaphoreType.DMA((2,)),
        ],
    )
    def copy_kernel(table_hbm, out_hbm, buf_a, buf_b, in_sems, out_sems):
        wid = lax.axis_index("s") * nc + lax.axis_index("c")
        base = wid * rows_per_w
        bufs = (buf_a, buf_b)

        in_cp = [None] * nchunks
        out_cp = [None] * nchunks
        for c, (off, n) in enumerate(sched):
            b = c % 2
            if c >= 2:
                out_cp[c - 2].wait()
            in_cp[c] = pltpu.async_copy(
                table_hbm.at[pl.ds(base + off, n)],
                bufs[b].at[pl.ds(0, n)],
                in_sems.at[b],
            )
            in_cp[c].wait()
            out_cp[c] = pltpu.async_copy(
                bufs[b].at[pl.ds(0, n)],
                out_hbm.at[pl.ds(base + off, n)],
                out_sems.at[b],
            )
        for c in range(max(0, nchunks - 2), nchunks):
            out_cp[c].wait()

    return copy_kernel


def kernel(x, emb_weight):
    seq = x.shape[1]
    return _make_copy(seq, emb_weight.shape[1], emb_weight.dtype.name)(emb_weight)


# speedup vs baseline: 1.0460x; 1.0460x over previous
import functools

import jax
import jax.numpy as jnp
from jax import lax
from jax.experimental import pallas as pl
from jax.experimental.pallas import tpu as pltpu
from jax.experimental.pallas import tpu_sc as plsc

_BUF_ROWS = (32, 24)


def _chunk_schedule(rows_per_w: int):
    sched = []
    off = 0
    while off < rows_per_w:
        n = min(_BUF_ROWS[len(sched) % 2], rows_per_w - off)
        sched.append((off, n))
        off += n
    return sched


@functools.lru_cache(maxsize=None)
def _make_copy(seq: int, d: int, dtype_name: str):
    dtype = jnp.dtype(dtype_name)
    info = plsc.get_sparse_core_info()
    nc, ns = info.num_cores, info.num_subcores
    nw = nc * ns
    assert seq % nw == 0
    rows_per_w = seq // nw
    sched = _chunk_schedule(rows_per_w)
    nchunks = len(sched)
    buf_rows = sum(_BUF_ROWS)

    mesh = plsc.VectorSubcoreMesh(core_axis_name="c", subcore_axis_name="s")

    @functools.partial(
        pl.kernel,
        mesh=mesh,
        out_type=jax.ShapeDtypeStruct((seq, d), dtype),
        scratch_types=[
            pltpu.MemorySpace.VMEM_SHARED((ns, buf_rows, d), dtype),
            pltpu.SemaphoreType.DMA((2,)),
            pltpu.SemaphoreType.DMA((2,)),
        ],
    )
    def copy_kernel(table_hbm, out_hbm, spmem, in_sems, out_sems):
        cid = lax.axis_index("c")
        sid = lax.axis_index("s")
        wid = sid * nc + cid
        base = wid * rows_per_w
        buf_off = (0, _BUF_ROWS[0])

        in_cp = [None] * nchunks
        out_cp = [None] * nchunks
        for c, (off, n) in enumerate(sched):
            b = c % 2
            if c >= 2:
                out_cp[c - 2].wait()
            in_cp[c] = pltpu.async_copy(
                table_hbm.at[pl.ds(base + off, n)],
                spmem.at[sid, pl.ds(buf_off[b], n)],
                in_sems.at[b],
            )
            in_cp[c].wait()
            out_cp[c] = pltpu.async_copy(
                spmem.at[sid, pl.ds(buf_off[b], n)],
                out_hbm.at[pl.ds(base + off, n)],
                out_sems.at[b],
            )
        for c in range(max(0, nchunks - 2), nchunks):
            out_cp[c].wait()

    return copy_kernel


def kernel(x, emb_weight):
    seq = x.shape[1]
    return _make_copy(seq, emb_weight.shape[1], emb_weight.dtype.name)(emb_weight)
